# baseline (speedup 1.0000x reference)
import jax
import jax.numpy as jnp
from jax import lax
from jax.experimental import pallas as pl
from jax.experimental.pallas import tpu as pltpu
from jax.experimental.pallas import tpu_sc as plsc

_N_ROWS = 4096
_N_FIELDS = 26
_FIELD_SIZE = 1000
_ROW_WORDS = _N_FIELDS * _FIELD_SIZE
_NUM_CORES = 2
_NUM_SUBCORES = 16
_NUM_WORKERS = _NUM_CORES * _NUM_SUBCORES
_ROWS_PER_W = _N_ROWS // _NUM_WORKERS
_CODES_PER_W = _ROWS_PER_W * _N_FIELDS
_NBUF = 4


def _sc_body(inp_ref, out_ref, b0, b1, b2, b3, codes, s0, s1, s2, s3):
    bufs = (b0, b1, b2, b3)
    sems = (s0, s1, s2, s3)
    wid = lax.axis_index("s") * _NUM_CORES + lax.axis_index("c")
    pltpu.sync_copy(inp_ref.at[pl.ds(wid * _CODES_PER_W, _CODES_PER_W)], codes)

    zeros = jnp.zeros((16,), jnp.float32)
    ones = jnp.ones((16,), jnp.float32)
    iota = lax.iota(jnp.int32, 16)
    mask_hi = iota >= 6

    for b in range(_NBUF):
        bb = bufs[b]

        def zero_body(i, carry, bb=bb):
            bb[pl.ds(i * 16, 16)] = zeros
            return carry

        lax.fori_loop(0, _ROW_WORDS // 16, zero_body, 0)

    def row_positions(rl):
        c0 = codes[pl.ds(rl * _N_FIELDS, 16)].astype(jnp.int32)
        c1 = codes[pl.ds(rl * _N_FIELDS + 10, 16)].astype(jnp.int32)
        pos0 = iota * _FIELD_SIZE + c0
        pos1 = (iota + 10) * _FIELD_SIZE + c1
        return pos0, pos1

    row_base = wid * _ROWS_PER_W

    def group_body(g, carry):
        for b in range(_NBUF):
            bb = bufs[b]
            rl = g * _NBUF + b
            row = row_base + rl

            @pl.when(g > 0)
            def _(bb=bb, rl=rl, row=row, b=b):
                old_dst = out_ref.at[
                    pl.ds((row - _NBUF) * _ROW_WORDS, _ROW_WORDS)
                ]
                pltpu.make_async_copy(bb, old_dst, sems[b]).wait()
                opos0, opos1 = row_positions(rl - _NBUF)
                plsc.store_scatter(bb, [opos0], zeros)
                plsc.store_scatter(bb, [opos1], zeros, mask=mask_hi)

            pos0, pos1 = row_positions(rl)
            plsc.store_scatter(bb, [pos0], ones)
            plsc.store_scatter(bb, [pos1], ones, mask=mask_hi)
            dst = out_ref.at[pl.ds(row * _ROW_WORDS, _ROW_WORDS)]
            pltpu.async_copy(bb, dst, sems[b])
        return carry

    lax.fori_loop(0, _ROWS_PER_W // _NBUF, group_body, 0)

    last_group = _ROWS_PER_W - _NBUF
    for b in range(_NBUF):
        row = row_base + last_group + b
        dst = out_ref.at[pl.ds(row * _ROW_WORDS, _ROW_WORDS)]
        pltpu.make_async_copy(bufs[b], dst, sems[b]).wait()


def kernel(input):
    n = input.shape[0]
    flat_in = input.reshape(-1)
    mesh = plsc.VectorSubcoreMesh(
        core_axis_name="c", subcore_axis_name="s"
    )
    out = pl.kernel(
        _sc_body,
        out_type=jax.ShapeDtypeStruct((n * _ROW_WORDS,), jnp.float32),
        mesh=mesh,
        compiler_params=pltpu.CompilerParams(needs_layout_passes=False),
        scratch_types=[
            pltpu.VMEM((_ROW_WORDS,), jnp.float32),
            pltpu.VMEM((_ROW_WORDS,), jnp.float32),
            pltpu.VMEM((_ROW_WORDS,), jnp.float32),
            pltpu.VMEM((_ROW_WORDS,), jnp.float32),
            pltpu.VMEM((_CODES_PER_W,), jnp.float32),
            pltpu.SemaphoreType.DMA,
            pltpu.SemaphoreType.DMA,
            pltpu.SemaphoreType.DMA,
            pltpu.SemaphoreType.DMA,
        ],
    )(flat_in)
    return out.reshape(n, _ROW_WORDS)


# speedup vs baseline: 2.1974x; 2.1974x over previous
import jax
import jax.numpy as jnp
from jax.experimental import pallas as pl

_N_FIELDS = 26
_FIELD_SIZE = 1000
_ROW_WORDS = _N_FIELDS * _FIELD_SIZE
_ROW_BLOCK = 128


def _onehot_block(in_ref, out_ref):
    codes = in_ref[...]
    offs = jax.lax.broadcasted_iota(
        jnp.int32, (_ROW_BLOCK, _FIELD_SIZE), 1
    ).astype(jnp.float32)
    for f in range(_N_FIELDS):
        out_ref[:, f * _FIELD_SIZE:(f + 1) * _FIELD_SIZE] = (
            offs == codes[:, f:f + 1]
        ).astype(jnp.float32)


def kernel(input):
    n = input.shape[0]
    grid = (n // _ROW_BLOCK,)
    return pl.pallas_call(
        _onehot_block,
        grid=grid,
        in_specs=[pl.BlockSpec((_ROW_BLOCK, _N_FIELDS), lambda r: (r, 0))],
        out_specs=pl.BlockSpec((_ROW_BLOCK, _ROW_WORDS), lambda r: (r, 0)),
        out_shape=jax.ShapeDtypeStruct((n, _ROW_WORDS), jnp.float32),
    )(input)
